# baseline (speedup 1.0000x reference)
import functools

import jax
import jax.numpy as jnp
from jax.experimental import pallas as pl
from jax.experimental.pallas import tpu as pltpu

_TAU = 0.5
_B = 2048
_CH = 512


def _body(bi_ref, bj_ref, h1_ref, h2_ref, outr_ref, outc_ref, dots_ref, w_ref,
          *, num_pairs, n, np_, th, sqrt_c):
    t = pl.program_id(0)
    bi = bi_ref[t]
    bj = bj_ref[t]
    is_diag = bi == bj
    dn = (((1,), (1,)), ((), ()))
    dn2 = (((1,), (0,)), ((), ()))

    @pl.when(t == 0)
    def _init():
        outr_ref[...] = jnp.zeros_like(outr_ref)
        outc_ref[...] = jnp.zeros_like(outc_ref)

    fill = jnp.logical_and(bi == 0, bj < th)

    @pl.when(fill)
    def _fill():
        h1b = h1_ref[...]
        h2b = h2_ref[...]
        rows = bj * _B + jax.lax.broadcasted_iota(jnp.int32, (_B, 128), 0)
        valid = rows < n
        h1m = jnp.where(valid, h1b, jnp.float32(0.0))
        h2m = jnp.where(valid, h2b, jnp.float32(0.0))
        ones_bf = jnp.ones((128, 128), jnp.bfloat16)
        ss1 = jax.lax.dot_general((h1m * h1m).astype(jnp.bfloat16), ones_bf,
                                  dn2, preferred_element_type=jnp.float32)
        ss2 = jax.lax.dot_general((h2m * h2m).astype(jnp.bfloat16), ones_bf,
                                  dn2, preferred_element_type=jnp.float32)
        r1 = jax.lax.rsqrt(ss1)
        r2 = jax.lax.rsqrt(ss2)
        sc = jnp.float32(sqrt_c)
        z1 = jnp.where(valid, h1m * r1 * sc, jnp.float32(0.0))
        z2 = jnp.where(valid, h2m * r2 * sc, jnp.float32(0.0))
        w_ref[pl.ds(bj * _B, _B), :] = z1.astype(jnp.bfloat16)
        w_ref[pl.ds(np_ + bj * _B, _B), :] = z2.astype(jnp.bfloat16)

        d12 = jax.lax.dot_general((h1m * h2m).astype(jnp.bfloat16), ones_bf,
                                  dn2, preferred_element_type=jnp.float32)
        zd = jnp.where(valid, d12 * r1 * r2, jnp.float32(0.0))
        acc8 = zd[0:8, :]
        for k in range(1, _B // 8):
            acc8 = acc8 + zd[k * 8 : (k + 1) * 8, :]

        @pl.when(t == 0)
        def _d0():
            dots_ref[...] = acc8

        @pl.when(t > 0)
        def _dacc():
            dots_ref[...] += acc8

    wj = w_ref[pl.ds(bj * _B, _B), :]

    col8 = jnp.zeros((8, _B), jnp.float32)
    n_ch = _B // _CH

    def _mm(r):
        wi_r = w_ref[pl.ds(bi * _B + r * _CH, _CH), :]
        return jax.lax.dot_general(wi_r, wj, dn, preferred_element_type=jnp.float32)

    s_cur = _mm(0)
    for r in range(n_ch):
        s_nxt = _mm(r + 1) if r + 1 < n_ch else None
        e_r = jnp.exp2(s_cur)
        rowp = e_r[:, 0:128]
        for g in range(1, _B // 128):
            rowp = rowp + e_r[:, g * 128 : (g + 1) * 128]
        outr_ref[pl.ds(bi, 1), pl.ds(r * _CH, _CH), :] += rowp[None]

        cpart = e_r[0:8, :]
        for k in range(1, _CH // 8):
            cpart = cpart + e_r[k * 8 : (k + 1) * 8, :]
        col8 = col8 + cpart
        s_cur = s_nxt

    @pl.when(jnp.logical_not(is_diag))
    def _col_acc():
        outc_ref[pl.ds(bj, 1)] += col8[None, :, :]


@functools.partial(jax.jit, static_argnums=(2, 3, 4))
def _fused(h1, h2, n, np_, sqrt_c):
    m = 2 * np_
    t_blocks = m // _B
    th = np_ // _B
    pairs = [(i, j) for i in range(t_blocks) for j in range(i, t_blocks)]
    num_pairs = len(pairs)
    bi_arr = jnp.asarray([p[0] for p in pairs], dtype=jnp.int32)
    bj_arr = jnp.asarray([p[1] for p in pairs], dtype=jnp.int32)
    d = h1.shape[1]

    def h_idx(t, bi, bj):
        return (jnp.where(bi[t] == 0, jnp.minimum(bj[t], th - 1), th - 1), 0)

    grid_spec = pltpu.PrefetchScalarGridSpec(
        num_scalar_prefetch=2,
        grid=(num_pairs,),
        in_specs=[
            pl.BlockSpec((_B, d), h_idx),
            pl.BlockSpec((_B, d), h_idx),
        ],
        out_specs=[
            pl.BlockSpec((t_blocks, _B, 128), lambda t, bi, bj: (0, 0, 0)),
            pl.BlockSpec((t_blocks, 8, _B), lambda t, bi, bj: (0, 0, 0)),
            pl.BlockSpec((8, 128), lambda t, bi, bj: (0, 0)),
        ],
        scratch_shapes=[pltpu.VMEM((m, d), jnp.bfloat16)],
    )
    outr, outc, dots = pl.pallas_call(
        functools.partial(_body, num_pairs=num_pairs, n=n, np_=np_, th=th,
                          sqrt_c=sqrt_c),
        grid_spec=grid_spec,
        out_shape=[
            jax.ShapeDtypeStruct((t_blocks, _B, 128), jnp.float32),
            jax.ShapeDtypeStruct((t_blocks, 8, _B), jnp.float32),
            jax.ShapeDtypeStruct((8, 128), jnp.float32),
        ],
        compiler_params=pltpu.CompilerParams(
            dimension_semantics=("arbitrary",),
        ),
    )(bi_arr, bj_arr, h1, h2)
    g = jnp.sum(outr, axis=2).reshape(m) + jnp.sum(outc, axis=1).reshape(m)
    return g, dots


def kernel(h1, h2):
    n, d = h1.shape
    inv_tau = jnp.float32(1.0 / _TAU)
    c = float(1.0 / _TAU) * 1.4426950408889634
    sqrt_c = c ** 0.5

    np_ = ((n + _B - 1) // _B) * _B
    pad = np_ - n

    g, dots = _fused(h1, h2, n, np_, sqrt_c)
    s1 = g[:n]
    s2 = g[np_ : np_ + n]

    self_sim = jnp.exp(inv_tau)
    pad_ones = jnp.float32(2 * pad)
    denom1 = s1 - pad_ones - self_sim
    denom2 = s2 - pad_ones - self_sim
    mean_log_pos = jnp.sum(dots) / (128.0 * n) * inv_tau
    return (jnp.mean(jnp.log(denom1)) + jnp.mean(jnp.log(denom2))) * jnp.float32(
        0.5
    ) - mean_log_pos


# speedup vs baseline: 1.0288x; 1.0288x over previous
import functools

import jax
import jax.numpy as jnp
from jax.experimental import pallas as pl
from jax.experimental.pallas import tpu as pltpu

_TAU = 0.5
_B = 2048
_CH = 512


def _body(bi_ref, bj_ref, h1_ref, h2_ref, outr_ref, outc_ref, dots_ref, w_ref,
          *, num_pairs, n, np_, th, sqrt_c):
    t = pl.program_id(0)
    bi = bi_ref[t]
    bj = bj_ref[t]
    is_diag = bi == bj
    dn = (((1,), (1,)), ((), ()))
    dn2 = (((1,), (0,)), ((), ()))

    @pl.when(t == 0)
    def _init():
        outr_ref[...] = jnp.zeros_like(outr_ref)
        outc_ref[...] = jnp.zeros_like(outc_ref)

    fill = jnp.logical_and(bi == 0, bj < th)

    @pl.when(fill)
    def _fill():
        h1b = h1_ref[...]
        h2b = h2_ref[...]
        rows = bj * _B + jax.lax.broadcasted_iota(jnp.int32, (_B, 128), 0)
        valid = rows < n
        h1m = jnp.where(valid, h1b, jnp.float32(0.0))
        h2m = jnp.where(valid, h2b, jnp.float32(0.0))
        ones_bf = jnp.ones((128, 128), jnp.bfloat16)
        ss1 = jax.lax.dot_general((h1m * h1m).astype(jnp.bfloat16), ones_bf,
                                  dn2, preferred_element_type=jnp.float32)
        ss2 = jax.lax.dot_general((h2m * h2m).astype(jnp.bfloat16), ones_bf,
                                  dn2, preferred_element_type=jnp.float32)
        r1 = jax.lax.rsqrt(ss1)
        r2 = jax.lax.rsqrt(ss2)
        sc = jnp.float32(sqrt_c)
        z1 = jnp.where(valid, h1m * r1 * sc, jnp.float32(0.0))
        z2 = jnp.where(valid, h2m * r2 * sc, jnp.float32(0.0))
        w_ref[pl.ds(bj * _B, _B), :] = z1.astype(jnp.float8_e4m3fn)
        w_ref[pl.ds(np_ + bj * _B, _B), :] = z2.astype(jnp.float8_e4m3fn)

        d12 = jax.lax.dot_general((h1m * h2m).astype(jnp.bfloat16), ones_bf,
                                  dn2, preferred_element_type=jnp.float32)
        zd = jnp.where(valid, d12 * r1 * r2, jnp.float32(0.0))
        acc8 = zd[0:8, :]
        for k in range(1, _B // 8):
            acc8 = acc8 + zd[k * 8 : (k + 1) * 8, :]

        @pl.when(t == 0)
        def _d0():
            dots_ref[...] = acc8

        @pl.when(t > 0)
        def _dacc():
            dots_ref[...] += acc8

    wj = w_ref[pl.ds(bj * _B, _B), :]

    col8 = jnp.zeros((8, _B), jnp.float32)
    n_ch = _B // _CH

    def _mm(r):
        wi_r = w_ref[pl.ds(bi * _B + r * _CH, _CH), :]
        return jax.lax.dot_general(wi_r, wj, dn, preferred_element_type=jnp.float32)

    s_cur = _mm(0)
    for r in range(n_ch):
        s_nxt = _mm(r + 1) if r + 1 < n_ch else None
        e_r = jnp.exp2(s_cur)
        rowp = e_r[:, 0:128]
        for g in range(1, _B // 128):
            rowp = rowp + e_r[:, g * 128 : (g + 1) * 128]
        outr_ref[pl.ds(bi, 1), pl.ds(r * _CH, _CH), :] += rowp[None]

        cpart = e_r[0:8, :]
        for k in range(1, _CH // 8):
            cpart = cpart + e_r[k * 8 : (k + 1) * 8, :]
        col8 = col8 + cpart
        s_cur = s_nxt

    @pl.when(jnp.logical_not(is_diag))
    def _col_acc():
        outc_ref[pl.ds(bj, 1)] += col8[None, :, :]


@functools.partial(jax.jit, static_argnums=(2, 3, 4))
def _fused(h1, h2, n, np_, sqrt_c):
    m = 2 * np_
    t_blocks = m // _B
    th = np_ // _B
    pairs = [(i, j) for i in range(t_blocks) for j in range(i, t_blocks)]
    num_pairs = len(pairs)
    bi_arr = jnp.asarray([p[0] for p in pairs], dtype=jnp.int32)
    bj_arr = jnp.asarray([p[1] for p in pairs], dtype=jnp.int32)
    d = h1.shape[1]

    def h_idx(t, bi, bj):
        return (jnp.where(bi[t] == 0, jnp.minimum(bj[t], th - 1), th - 1), 0)

    grid_spec = pltpu.PrefetchScalarGridSpec(
        num_scalar_prefetch=2,
        grid=(num_pairs,),
        in_specs=[
            pl.BlockSpec((_B, d), h_idx),
            pl.BlockSpec((_B, d), h_idx),
        ],
        out_specs=[
            pl.BlockSpec((t_blocks, _B, 128), lambda t, bi, bj: (0, 0, 0)),
            pl.BlockSpec((t_blocks, 8, _B), lambda t, bi, bj: (0, 0, 0)),
            pl.BlockSpec((8, 128), lambda t, bi, bj: (0, 0)),
        ],
        scratch_shapes=[pltpu.VMEM((m, d), jnp.float8_e4m3fn)],
    )
    outr, outc, dots = pl.pallas_call(
        functools.partial(_body, num_pairs=num_pairs, n=n, np_=np_, th=th,
                          sqrt_c=sqrt_c),
        grid_spec=grid_spec,
        out_shape=[
            jax.ShapeDtypeStruct((t_blocks, _B, 128), jnp.float32),
            jax.ShapeDtypeStruct((t_blocks, 8, _B), jnp.float32),
            jax.ShapeDtypeStruct((8, 128), jnp.float32),
        ],
        compiler_params=pltpu.CompilerParams(
            dimension_semantics=("arbitrary",),
        ),
    )(bi_arr, bj_arr, h1, h2)
    g = jnp.sum(outr, axis=2).reshape(m) + jnp.sum(outc, axis=1).reshape(m)
    return g, dots


def kernel(h1, h2):
    n, d = h1.shape
    inv_tau = jnp.float32(1.0 / _TAU)
    c = float(1.0 / _TAU) * 1.4426950408889634
    sqrt_c = c ** 0.5

    np_ = ((n + _B - 1) // _B) * _B
    pad = np_ - n

    g, dots = _fused(h1, h2, n, np_, sqrt_c)
    s1 = g[:n]
    s2 = g[np_ : np_ + n]

    self_sim = jnp.exp(inv_tau)
    pad_ones = jnp.float32(2 * pad)
    denom1 = s1 - pad_ones - self_sim
    denom2 = s2 - pad_ones - self_sim
    mean_log_pos = jnp.sum(dots) / (128.0 * n) * inv_tau
    return (jnp.mean(jnp.log(denom1)) + jnp.mean(jnp.log(denom2))) * jnp.float32(
        0.5
    ) - mean_log_pos


# speedup vs baseline: 1.0333x; 1.0044x over previous
import functools

import jax
import jax.numpy as jnp
from jax.experimental import pallas as pl
from jax.experimental.pallas import tpu as pltpu

_TAU = 0.5
_B = 2560
_CH = 512


def _body(bi_ref, bj_ref, h1_ref, h2_ref, outr_ref, outc_ref, dots_ref, w_ref,
          *, num_pairs, n, np_, th, sqrt_c):
    t = pl.program_id(0)
    bi = bi_ref[t]
    bj = bj_ref[t]
    is_diag = bi == bj
    dn = (((1,), (1,)), ((), ()))
    dn2 = (((1,), (0,)), ((), ()))

    @pl.when(t == 0)
    def _init():
        outr_ref[...] = jnp.zeros_like(outr_ref)
        outc_ref[...] = jnp.zeros_like(outc_ref)

    fill = jnp.logical_and(bi == 0, bj < th)

    @pl.when(fill)
    def _fill():
        h1b = h1_ref[...]
        h2b = h2_ref[...]
        rows = bj * _B + jax.lax.broadcasted_iota(jnp.int32, (_B, 128), 0)
        valid = rows < n
        h1m = jnp.where(valid, h1b, jnp.float32(0.0))
        h2m = jnp.where(valid, h2b, jnp.float32(0.0))
        ones_bf = jnp.ones((128, 128), jnp.bfloat16)
        ss1 = jax.lax.dot_general((h1m * h1m).astype(jnp.bfloat16), ones_bf,
                                  dn2, preferred_element_type=jnp.float32)
        ss2 = jax.lax.dot_general((h2m * h2m).astype(jnp.bfloat16), ones_bf,
                                  dn2, preferred_element_type=jnp.float32)
        r1 = jax.lax.rsqrt(ss1)
        r2 = jax.lax.rsqrt(ss2)
        sc = jnp.float32(sqrt_c)
        z1 = jnp.where(valid, h1m * r1 * sc, jnp.float32(0.0))
        z2 = jnp.where(valid, h2m * r2 * sc, jnp.float32(0.0))
        w_ref[pl.ds(bj * _B, _B), :] = z1.astype(jnp.float8_e4m3fn)
        w_ref[pl.ds(np_ + bj * _B, _B), :] = z2.astype(jnp.float8_e4m3fn)

        d12 = jax.lax.dot_general((h1m * h2m).astype(jnp.bfloat16), ones_bf,
                                  dn2, preferred_element_type=jnp.float32)
        zd = jnp.where(valid, d12 * r1 * r2, jnp.float32(0.0))
        acc8 = zd[0:8, :]
        for k in range(1, _B // 8):
            acc8 = acc8 + zd[k * 8 : (k + 1) * 8, :]

        @pl.when(t == 0)
        def _d0():
            dots_ref[...] = acc8

        @pl.when(t > 0)
        def _dacc():
            dots_ref[...] += acc8

    wj = w_ref[pl.ds(bj * _B, _B), :]

    col8 = jnp.zeros((8, _B), jnp.float32)
    n_ch = _B // _CH

    def _mm(r):
        wi_r = w_ref[pl.ds(bi * _B + r * _CH, _CH), :]
        return jax.lax.dot_general(wi_r, wj, dn, preferred_element_type=jnp.float32)

    s_cur = _mm(0)
    for r in range(n_ch):
        s_nxt = _mm(r + 1) if r + 1 < n_ch else None
        e_r = jnp.exp2(s_cur)
        rowp = e_r[:, 0:128]
        for g in range(1, _B // 128):
            rowp = rowp + e_r[:, g * 128 : (g + 1) * 128]
        outr_ref[pl.ds(bi, 1), pl.ds(r * _CH, _CH), :] += rowp[None]

        cpart = e_r[0:8, :]
        for k in range(1, _CH // 8):
            cpart = cpart + e_r[k * 8 : (k + 1) * 8, :]
        col8 = col8 + cpart
        s_cur = s_nxt

    @pl.when(jnp.logical_not(is_diag))
    def _col_acc():
        outc_ref[pl.ds(bj, 1)] += col8[None, :, :]


@functools.partial(jax.jit, static_argnums=(2, 3, 4))
def _fused(h1, h2, n, np_, sqrt_c):
    m = 2 * np_
    t_blocks = m // _B
    th = np_ // _B
    pairs = [(i, j) for i in range(t_blocks) for j in range(i, t_blocks)]
    num_pairs = len(pairs)
    bi_arr = jnp.asarray([p[0] for p in pairs], dtype=jnp.int32)
    bj_arr = jnp.asarray([p[1] for p in pairs], dtype=jnp.int32)
    d = h1.shape[1]

    def h_idx(t, bi, bj):
        return (jnp.where(bi[t] == 0, jnp.minimum(bj[t], th - 1), th - 1), 0)

    grid_spec = pltpu.PrefetchScalarGridSpec(
        num_scalar_prefetch=2,
        grid=(num_pairs,),
        in_specs=[
            pl.BlockSpec((_B, d), h_idx),
            pl.BlockSpec((_B, d), h_idx),
        ],
        out_specs=[
            pl.BlockSpec((t_blocks, _B, 128), lambda t, bi, bj: (0, 0, 0)),
            pl.BlockSpec((t_blocks, 8, _B), lambda t, bi, bj: (0, 0, 0)),
            pl.BlockSpec((8, 128), lambda t, bi, bj: (0, 0)),
        ],
        scratch_shapes=[pltpu.VMEM((m, d), jnp.float8_e4m3fn)],
    )
    outr, outc, dots = pl.pallas_call(
        functools.partial(_body, num_pairs=num_pairs, n=n, np_=np_, th=th,
                          sqrt_c=sqrt_c),
        grid_spec=grid_spec,
        out_shape=[
            jax.ShapeDtypeStruct((t_blocks, _B, 128), jnp.float32),
            jax.ShapeDtypeStruct((t_blocks, 8, _B), jnp.float32),
            jax.ShapeDtypeStruct((8, 128), jnp.float32),
        ],
        compiler_params=pltpu.CompilerParams(
            dimension_semantics=("arbitrary",),
        ),
    )(bi_arr, bj_arr, h1, h2)
    g = jnp.sum(outr, axis=2).reshape(m) + jnp.sum(outc, axis=1).reshape(m)
    return g, dots


def kernel(h1, h2):
    n, d = h1.shape
    inv_tau = jnp.float32(1.0 / _TAU)
    c = float(1.0 / _TAU) * 1.4426950408889634
    sqrt_c = c ** 0.5

    np_ = ((n + _B - 1) // _B) * _B
    pad = np_ - n

    g, dots = _fused(h1, h2, n, np_, sqrt_c)
    s1 = g[:n]
    s2 = g[np_ : np_ + n]

    self_sim = jnp.exp(inv_tau)
    pad_ones = jnp.float32(2 * pad)
    denom1 = s1 - pad_ones - self_sim
    denom2 = s2 - pad_ones - self_sim
    mean_log_pos = jnp.sum(dots) / (128.0 * n) * inv_tau
    return (jnp.mean(jnp.log(denom1)) + jnp.mean(jnp.log(denom2))) * jnp.float32(
        0.5
    ) - mean_log_pos
